# baseline (speedup 1.0000x reference)
import functools

import jax
import jax.numpy as jnp
from jax import lax
from jax.experimental import pallas as pl
from jax.experimental.pallas import tpu as pltpu
from jax.experimental.pallas import tpu_sc as plsc

VOCAB = 1000000
EMBD = 64
B = 16384
PACK = 128
VH = VOCAB // 2

NC = 2
NS = 16
L = 16
NW = NC * NS
BPW = B // NW
CHUNK = 128
NCH = BPW // CHUNK
GPC = CHUNK // L

_mesh = plsc.VectorSubcoreMesh(core_axis_name="c", subcore_axis_name="s")


@functools.partial(
    pl.kernel,
    out_type=jax.ShapeDtypeStruct((NW, BPW), jnp.float32),
    mesh=_mesh,
    compiler_params=pltpu.CompilerParams(needs_layout_passes=False),
    scratch_types=[
        pltpu.VMEM((NCH, CHUNK), jnp.int32),
        pltpu.VMEM((NCH, CHUNK), jnp.int32),
        pltpu.VMEM((NCH, CHUNK), jnp.int32),
        pltpu.VMEM((NCH, CHUNK), jnp.int32),
        pltpu.VMEM((2, CHUNK, PACK), jnp.float32),
        pltpu.VMEM((2, CHUNK, PACK), jnp.float32),
        pltpu.VMEM((BPW,), jnp.float32),
        pltpu.SemaphoreType.DMA,
        pltpu.SemaphoreType.DMA,
    ],
)
def _skipgram_sc(focus_hbm, context_hbm, emb_hbm, out_hbm,
                 fidx, cidx, fpid, cpid, frows, crows, outv, sem0, sem1):
    wid = lax.axis_index("s") * NC + lax.axis_index("c")

    pltpu.sync_copy(focus_hbm.at[wid], fidx)
    pltpu.sync_copy(context_hbm.at[wid], cidx)

    vh = jnp.full((L,), VH, jnp.int32)
    for j in range(NCH):
        for k in range(CHUNK // L):
            sl = pl.ds(k * L, L)
            fv = fidx.at[j][sl]
            cv = cidx.at[j][sl]
            fpid.at[j][sl] = jnp.where(fv >= vh, fv - vh, fv)
            cpid.at[j][sl] = jnp.where(cv >= vh, cv - vh, cv)

    iota = lax.iota(jnp.int32, L)
    sems = [sem0, sem1]

    def gather(j, p):
        return [
            pltpu.async_copy(emb_hbm.at[fpid.at[j]], frows.at[p], sems[p]),
            pltpu.async_copy(emb_hbm.at[cpid.at[j]], crows.at[p], sems[p]),
        ]

    def compute(j, p):
        def body(g, _):
            row = g * L + iota
            sl = pl.ds(g * L, L)
            fv = fidx.at[j][sl]
            cv = cidx.at[j][sl]
            colf = jnp.where(fv >= vh, EMBD, 0).astype(jnp.int32)
            colc = jnp.where(cv >= vh, EMBD, 0).astype(jnp.int32)
            acc = jnp.zeros((L,), jnp.float32)
            for d in range(EMBD):
                f = plsc.load_gather(frows.at[p], [row, colf])
                c = plsc.load_gather(crows.at[p], [row, colc])
                acc = acc + f * c
                if d != EMBD - 1:
                    colf = colf + 1
                    colc = colc + 1
            outv[pl.ds(j * CHUNK + g * L, L)] = acc
            return _

        lax.fori_loop(0, GPC, body, None)

    pend = gather(0, 0)
    for j in range(NCH):
        p = j % 2
        nxt = gather(j + 1, (j + 1) % 2) if j + 1 < NCH else []
        for c in pend:
            c.wait()
        compute(j, p)
        pend = nxt

    pltpu.sync_copy(outv, out_hbm.at[wid])


def kernel(focus, context, embeddings):
    et = embeddings.T
    packed = et.reshape(EMBD, 2, VH).transpose(2, 1, 0).reshape(VH, PACK)
    focus = focus.reshape(NW, NCH, CHUNK)
    context = context.reshape(NW, NCH, CHUNK)
    out = _skipgram_sc(focus, context, packed)
    return out.reshape(B)


# speedup vs baseline: 3.5679x; 3.5679x over previous
import functools

import jax
import jax.numpy as jnp
from jax import lax
from jax.experimental import pallas as pl
from jax.experimental.pallas import tpu as pltpu
from jax.experimental.pallas import tpu_sc as plsc

VOCAB = 1000000
EMBD = 64
B = 16384
SUB = 8

NC = 2
NS = 16
L = 16
NW = NC * NS
BPW = B // NW
GROUPS = BPW // L
PAIRS = GROUPS // 2

_mesh = plsc.VectorSubcoreMesh(core_axis_name="c", subcore_axis_name="s")


@functools.partial(
    pl.kernel,
    out_type=jax.ShapeDtypeStruct((NW, BPW), jnp.float32),
    mesh=_mesh,
    compiler_params=pltpu.CompilerParams(needs_layout_passes=False),
    scratch_types=[
        pltpu.VMEM((GROUPS, L), jnp.int32),
        pltpu.VMEM((GROUPS, L), jnp.int32),
        pltpu.VMEM((2, L, SUB, EMBD), jnp.float32),
        pltpu.VMEM((2, L, SUB, EMBD), jnp.float32),
        pltpu.VMEM((L * L,), jnp.float32),
        pltpu.VMEM((BPW,), jnp.float32),
        pltpu.SemaphoreType.DMA,
        pltpu.SemaphoreType.DMA,
    ],
)
def _skipgram_sc(focus_hbm, context_hbm, emb_hbm, out_hbm,
                 fidx, cidx, fblk, cblk, part, outv, sem0, sem1):
    wid = lax.axis_index("s") * NC + lax.axis_index("c")

    pltpu.sync_copy(focus_hbm.at[wid], fidx)
    pltpu.sync_copy(context_hbm.at[wid], cidx)

    iota = lax.iota(jnp.int32, L)
    col_base = iota * L
    sems = [sem0, sem1]

    def enqueue(g, p):
        fv = fidx[g]
        cv = cidx[g]
        fa = fv & ~(SUB - 1)
        ca = cv & ~(SUB - 1)
        for rr in range(L):
            pltpu.async_copy(
                emb_hbm.at[pl.ds(pl.multiple_of(fa[rr], SUB), SUB), :],
                fblk.at[p].at[rr], sems[p])
            pltpu.async_copy(
                emb_hbm.at[pl.ds(pl.multiple_of(ca[rr], SUB), SUB), :],
                cblk.at[p].at[rr], sems[p])

    def drain(p):
        for rr in range(L):
            pltpu.make_async_copy(
                emb_hbm.at[pl.ds(0, SUB), :], fblk.at[p].at[rr],
                sems[p]).wait()
            pltpu.make_async_copy(
                emb_hbm.at[pl.ds(0, SUB), :], cblk.at[p].at[rr],
                sems[p]).wait()

    def compute(g, p):
        fv = fidx[g]
        cv = cidx[g]
        fs = fv & (SUB - 1)
        cs = cv & (SUB - 1)
        for rr in range(L):
            rowf = jnp.full((L,), fs[rr], jnp.int32)
            rowc = jnp.full((L,), cs[rr], jnp.int32)
            acc = None
            for k in range(EMBD // L):
                col = k * L + iota
                f = plsc.load_gather(fblk.at[p].at[rr], [rowf, col])
                c = plsc.load_gather(cblk.at[p].at[rr], [rowc, col])
                acc = f * c if acc is None else acc + f * c
            plsc.store_scatter(part, [col_base + rr], acc)
        tot = part[pl.ds(0, L)]
        for cix in range(1, L):
            tot = tot + part[pl.ds(cix * L, L)]
        outv[pl.ds(g * L, L)] = tot

    enqueue(0, 0)

    def body(i, _):
        g0 = 2 * i
        enqueue(g0 + 1, 1)
        drain(0)
        compute(g0, 0)

        @pl.when(i + 1 < PAIRS)
        def _():
            enqueue(g0 + 2, 0)

        drain(1)
        compute(g0 + 1, 1)
        return _

    lax.fori_loop(0, PAIRS, body, None)

    pltpu.sync_copy(outv, out_hbm.at[wid])


def kernel(focus, context, embeddings):
    focus = focus.reshape(NW, GROUPS, L)
    context = context.reshape(NW, GROUPS, L)
    out = _skipgram_sc(focus, context, embeddings)
    return out.reshape(B)
